# baseline (speedup 1.0000x reference)
import functools

import jax
import jax.numpy as jnp
from jax import lax
from jax.experimental import pallas as pl
from jax.experimental.pallas import tpu as pltpu
from jax.experimental.pallas import tpu_sc as plsc

VOCAB = 30522
EMBED_DIM = 128
NUM_CLASS = 3
BATCH = 16384
SEQ = 200

VPAD = 30720
VBLK = 15360

NC = 2
NS = 16
NW = NC * NS
BPW = BATCH // NW
CHUNK = 128
NCH = BPW // CHUNK
LG = CHUNK // 16
UNROLL = 8


def _proj_body(w_ref, b_ref, emb_ref, q01_ref, p2_ref):
    scale = jnp.float32(1.0 / SEQ)
    d = lax.dot_general(
        w_ref[...] * scale, emb_ref[...], (((1,), (1,)), ((), ())),
        preferred_element_type=jnp.float32) + b_ref[...] * scale
    r0 = lax.bitcast_convert_type(d[0:1], jnp.uint32) + jnp.uint32(0x8000)
    r1 = lax.bitcast_convert_type(d[1:2], jnp.uint32) + jnp.uint32(0x8000)
    q = (r0 >> 16) | (r1 & jnp.uint32(0xFFFF0000))
    q01_ref[...] = lax.bitcast_convert_type(q, jnp.int32)
    p2_ref[...] = d[2:3]


def _project_table(w8, b8, emb_table):
    return pl.pallas_call(
        _proj_body,
        grid=(VPAD // VBLK,),
        in_specs=[
            pl.BlockSpec((NUM_CLASS, EMBED_DIM), lambda i: (0, 0)),
            pl.BlockSpec((NUM_CLASS, 1), lambda i: (0, 0)),
            pl.BlockSpec((VBLK, EMBED_DIM), lambda i: (i, 0)),
        ],
        out_specs=[
            pl.BlockSpec((1, VBLK), lambda i: (0, i)),
            pl.BlockSpec((1, VBLK), lambda i: (0, i)),
        ],
        out_shape=[
            jax.ShapeDtypeStruct((1, VPAD), jnp.int32),
            jax.ShapeDtypeStruct((1, VPAD), jnp.float32),
        ],
    )(w8, b8, emb_table)


def _sc_body(q01_hbm, p2_hbm, texts_hbm, out_hbm,
             t01, t2, ib0, ib1, o0, o1, o2, s0, s1, st):
    cid = lax.axis_index("c")
    sid = lax.axis_index("s")
    wid = sid * NC + cid
    base = wid * BPW

    ibufs = (ib0, ib1)
    sems = (s0, s1)

    def fetch(k):
        return pltpu.async_copy(
            texts_hbm.at[:, pl.ds(base + k * CHUNK, CHUNK)],
            ibufs[k % 2], sems[k % 2])

    pending = fetch(0)
    pltpu.async_copy(q01_hbm.at[0], t01, st).wait()
    pltpu.async_copy(p2_hbm.at[0], t2, st).wait()

    zero = jnp.zeros((16,), jnp.float32)
    hi = jnp.full((16,), -65536, jnp.int32)

    for k in range(NCH):
        nxt = fetch(k + 1) if k + 1 < NCH else None
        pending.wait()
        pending = nxt
        ib = ibufs[k % 2]

        def lgroup(l, _):
            col = l * 16

            @plsc.parallel_loop(0, SEQ, UNROLL, unroll=2, carry=(zero,) * 6)
            def _accs(tb, accs):
                a0, a1, a2, c0, c1, c2 = accs
                for u in range(UNROLL):
                    ix = ib[tb + u, pl.ds(col, 16)]
                    q = plsc.load_gather(t01, [ix])
                    v2 = plsc.load_gather(t2, [ix])
                    v0 = plsc.bitcast(lax.shift_left(q, 16), jnp.float32)
                    v1 = plsc.bitcast(lax.bitwise_and(q, hi), jnp.float32)
                    if u % 2 == 0:
                        a0 = a0 + v0
                        a1 = a1 + v1
                        a2 = a2 + v2
                    else:
                        c0 = c0 + v0
                        c1 = c1 + v1
                        c2 = c2 + v2
                return a0, a1, a2, c0, c1, c2

            a0, a1, a2, c0, c1, c2 = _accs

            o0[0, pl.ds(k * CHUNK + col, 16)] = a0 + c0
            o1[0, pl.ds(k * CHUNK + col, 16)] = a1 + c1
            o2[0, pl.ds(k * CHUNK + col, 16)] = a2 + c2
            return 0

        lax.fori_loop(0, LG, lgroup, 0)

    pltpu.sync_copy(o0, out_hbm.at[pl.ds(0, 1), pl.ds(base, BPW)])
    pltpu.sync_copy(o1, out_hbm.at[pl.ds(1, 1), pl.ds(base, BPW)])
    pltpu.sync_copy(o2, out_hbm.at[pl.ds(2, 1), pl.ds(base, BPW)])


@functools.cache
def _sc_gather():
    return pl.kernel(
        _sc_body,
        out_type=jax.ShapeDtypeStruct((NUM_CLASS, BATCH), jnp.float32),
        mesh=plsc.VectorSubcoreMesh(core_axis_name="c", subcore_axis_name="s",
                                    num_cores=NC, num_subcores=NS),
        compiler_params=pltpu.CompilerParams(needs_layout_passes=False),
        scratch_types=[
            pltpu.VMEM((VPAD,), jnp.int32),
            pltpu.VMEM((VPAD,), jnp.float32),
            pltpu.VMEM((SEQ, CHUNK), jnp.int32),
            pltpu.VMEM((SEQ, CHUNK), jnp.int32),
            pltpu.VMEM((1, BPW), jnp.float32),
            pltpu.VMEM((1, BPW), jnp.float32),
            pltpu.VMEM((1, BPW), jnp.float32),
            pltpu.SemaphoreType.DMA,
            pltpu.SemaphoreType.DMA,
            pltpu.SemaphoreType.DMA,
        ],
    )


def kernel(texts, emb_table, lin_w, lin_b):
    q01, p2 = _project_table(lin_w, lin_b.reshape(NUM_CLASS, 1), emb_table)
    out = _sc_gather()(q01, p2, texts.T)
    return out.T


# speedup vs baseline: 1.0056x; 1.0056x over previous
import functools

import jax
import jax.numpy as jnp
from jax import lax
from jax.experimental import pallas as pl
from jax.experimental.pallas import tpu as pltpu
from jax.experimental.pallas import tpu_sc as plsc

VOCAB = 30522
EMBED_DIM = 128
NUM_CLASS = 3
BATCH = 16384
SEQ = 200

VPAD = 30720
VBLK = 15360

NC = 2
NS = 16
NW = NC * NS
BPW = BATCH // NW
CHUNK = 128
NCH = BPW // CHUNK
LG = CHUNK // 16
UNROLL = 8


def _proj_body(w_ref, b_ref, emb_ref, q01_ref, p2_ref):
    scale = jnp.float32(1.0 / SEQ)
    d = lax.dot_general(
        w_ref[...] * scale, emb_ref[...], (((1,), (1,)), ((), ())),
        preferred_element_type=jnp.float32) + b_ref[...] * scale
    r0 = lax.bitcast_convert_type(d[0:1], jnp.uint32) + jnp.uint32(0x8000)
    r1 = lax.bitcast_convert_type(d[1:2], jnp.uint32) + jnp.uint32(0x8000)
    q = (r0 >> 16) | (r1 & jnp.uint32(0xFFFF0000))
    q01_ref[...] = lax.bitcast_convert_type(q, jnp.int32)
    p2_ref[...] = d[2:3]


def _project_table(w8, b8, emb_table):
    return pl.pallas_call(
        _proj_body,
        grid=(VPAD // VBLK,),
        in_specs=[
            pl.BlockSpec((NUM_CLASS, EMBED_DIM), lambda i: (0, 0)),
            pl.BlockSpec((NUM_CLASS, 1), lambda i: (0, 0)),
            pl.BlockSpec((VBLK, EMBED_DIM), lambda i: (i, 0)),
        ],
        out_specs=[
            pl.BlockSpec((1, VBLK), lambda i: (0, i)),
            pl.BlockSpec((1, VBLK), lambda i: (0, i)),
        ],
        out_shape=[
            jax.ShapeDtypeStruct((1, VPAD), jnp.int32),
            jax.ShapeDtypeStruct((1, VPAD), jnp.float32),
        ],
    )(w8, b8, emb_table)


def _sc_body(q01_hbm, p2_hbm, texts_hbm, out_hbm,
             t01, t2, ib0, ib1, o0, o1, o2, s0, s1, st):
    cid = lax.axis_index("c")
    sid = lax.axis_index("s")
    wid = sid * NC + cid
    base = wid * BPW

    ibufs = (ib0, ib1)
    sems = (s0, s1)

    def fetch(k):
        return pltpu.async_copy(
            texts_hbm.at[:, pl.ds(base + k * CHUNK, CHUNK)],
            ibufs[k % 2], sems[k % 2])

    pending = fetch(0)
    pltpu.async_copy(q01_hbm.at[0], t01, st).wait()
    pltpu.async_copy(p2_hbm.at[0], t2, st).wait()

    zero = jnp.zeros((16,), jnp.float32)
    hi = jnp.full((16,), -65536, jnp.int32)

    for k in range(NCH):
        nxt = fetch(k + 1) if k + 1 < NCH else None
        pending.wait()
        pending = nxt
        ib = ibufs[k % 2]

        def lgroup(l, _):
            col = l * 16

            @plsc.parallel_loop(0, SEQ, UNROLL, carry=(zero,) * 6)
            def _accs(tb, accs):
                a0, a1, a2, c0, c1, c2 = accs
                for u in range(UNROLL):
                    ix = ib[tb + u, pl.ds(col, 16)]
                    q = plsc.load_gather(t01, [ix])
                    v2 = plsc.load_gather(t2, [ix])
                    v0 = plsc.bitcast(lax.shift_left(q, 16), jnp.float32)
                    v1 = plsc.bitcast(lax.bitwise_and(q, hi), jnp.float32)
                    if u % 2 == 0:
                        a0 = a0 + v0
                        a1 = a1 + v1
                        a2 = a2 + v2
                    else:
                        c0 = c0 + v0
                        c1 = c1 + v1
                        c2 = c2 + v2
                return a0, a1, a2, c0, c1, c2

            a0, a1, a2, c0, c1, c2 = _accs

            o0[0, pl.ds(k * CHUNK + col, 16)] = a0 + c0
            o1[0, pl.ds(k * CHUNK + col, 16)] = a1 + c1
            o2[0, pl.ds(k * CHUNK + col, 16)] = a2 + c2
            return 0

        lax.fori_loop(0, LG, lgroup, 0)

    pltpu.sync_copy(o0, out_hbm.at[pl.ds(0, 1), pl.ds(base, BPW)])
    pltpu.sync_copy(o1, out_hbm.at[pl.ds(1, 1), pl.ds(base, BPW)])
    pltpu.sync_copy(o2, out_hbm.at[pl.ds(2, 1), pl.ds(base, BPW)])


@functools.cache
def _sc_gather():
    return pl.kernel(
        _sc_body,
        out_type=jax.ShapeDtypeStruct((NUM_CLASS, BATCH), jnp.float32),
        mesh=plsc.VectorSubcoreMesh(core_axis_name="c", subcore_axis_name="s",
                                    num_cores=NC, num_subcores=NS),
        compiler_params=pltpu.CompilerParams(needs_layout_passes=False),
        scratch_types=[
            pltpu.VMEM((VPAD,), jnp.int32),
            pltpu.VMEM((VPAD,), jnp.float32),
            pltpu.VMEM((SEQ, CHUNK), jnp.int32),
            pltpu.VMEM((SEQ, CHUNK), jnp.int32),
            pltpu.VMEM((1, BPW), jnp.float32),
            pltpu.VMEM((1, BPW), jnp.float32),
            pltpu.VMEM((1, BPW), jnp.float32),
            pltpu.SemaphoreType.DMA,
            pltpu.SemaphoreType.DMA,
            pltpu.SemaphoreType.DMA,
        ],
    )


def kernel(texts, emb_table, lin_w, lin_b):
    q01, p2 = _project_table(lin_w, lin_b.reshape(NUM_CLASS, 1), emb_table)
    out = _sc_gather()(q01, p2, texts.T)
    return out.T
